# baseline (speedup 1.0000x reference)
import functools

import numpy as np

import jax
import jax.numpy as jnp
from jax import lax
from jax.experimental import pallas as pl
from jax.experimental.pallas import tpu as pltpu
from jax.experimental.pallas import tpu_sc as plsc

N = 10000
NPAD = 10240
E = 320000
EPAD = 327680
D = 128
H = 64

NC = 2
NS = 16
NW = NC * NS
EPW = EPAD // NW
CH = 128
NCHUNK = EPW // CH
RPW = NPAD // NS
NBUF = 8
LOOKAHEAD = 4
NPAIR = NPAD // 2

_P = np.arange(NPAD)
_MIDX = ((_P // 16) * 1024 + ((_P % 16) // 2) * 128 + (_P % 2)).astype(np.int32)
_MIDX = (_MIDX[None, :] + (np.arange(NC) * (NPAD // 2) * 128)[:, None]
         ).reshape(NC, NPAD // 128, 128).astype(np.int32)

_MESH = plsc.VectorSubcoreMesh(
    core_axis_name="c", subcore_axis_name="s", num_cores=NC, num_subcores=NS
)


def _deg_body(dst_hbm, zeros_hbm, midx_hbm, out_hbm,
              deg_sp, didx_v, ones_v, dval_v, midx_v, ssem):
    c = lax.axis_index("c")
    s = lax.axis_index("s")
    for j in range(8):
        ones_v[pl.ds(j * 16, 16)] = jnp.ones((16,), jnp.float32)
    pltpu.sync_copy(dst_hbm.at[pl.ds((c * NS + s) * NCHUNK, NCHUNK)], didx_v)
    pltpu.sync_copy(zeros_hbm.at[pl.ds(s * RPW, RPW)],
                    deg_sp.at[pl.ds(s * RPW, RPW)])
    plsc.subcore_barrier()

    def body(i, carry):
        pltpu.async_copy(ones_v, deg_sp.at[didx_v.at[i]], ssem, add=True)
        return carry

    lax.fori_loop(0, NCHUNK, body, 0)

    def drain(i, carry):
        pltpu.make_async_copy(ones_v, deg_sp.at[didx_v.at[0]], ssem).wait()
        return carry

    lax.fori_loop(0, NCHUNK, drain, 0)
    plsc.subcore_barrier()
    pltpu.sync_copy(deg_sp.at[pl.ds(s * RPW, RPW)], dval_v)
    pltpu.sync_copy(midx_hbm.at[c, pl.ds(s * (RPW // CH), RPW // CH)], midx_v)
    for k in range(RPW // CH):
        pltpu.sync_copy(dval_v.at[pl.ds(k * CH, CH)],
                        out_hbm.at[midx_v.at[k]])


_SC_PARAMS = pltpu.CompilerParams(use_tc_tiling_on_sc=False)

_deg_kernel = pl.kernel(
    _deg_body,
    out_type=jax.ShapeDtypeStruct((NC * NPAIR * 128,), jnp.float32),
    mesh=_MESH,
    compiler_params=_SC_PARAMS,
    scratch_types=[
        pltpu.VMEM_SHARED((NPAD,), jnp.float32),
        pltpu.VMEM((NCHUNK, CH), jnp.int32),
        pltpu.VMEM((CH,), jnp.float32),
        pltpu.VMEM((RPW,), jnp.float32),
        pltpu.VMEM((RPW // CH, CH), jnp.int32),
        pltpu.SemaphoreType.DMA,
    ],
)


def _seg_body(tab_hbm, src_hbm, dst_hbm, zeros_hbm, out_hbm,
              acc_sp, sidx_v, didx_v, rows_v, gsem, ssem):
    c = lax.axis_index("c")
    s = lax.axis_index("s")
    w = c * NS + s
    pltpu.sync_copy(src_hbm.at[pl.ds(w * NCHUNK, NCHUNK)], sidx_v)
    pltpu.sync_copy(dst_hbm.at[pl.ds(w * NCHUNK, NCHUNK)], didx_v)
    pltpu.sync_copy(zeros_hbm.at[pl.ds(s * RPW, RPW)],
                    acc_sp.at[pl.ds(s * RPW, RPW)])
    plsc.subcore_barrier()

    def wait_g(b):
        pltpu.make_async_copy(tab_hbm.at[sidx_v.at[0]], rows_v.at[b],
                              gsem).wait()

    def wait_s():
        pltpu.make_async_copy(rows_v.at[0], acc_sp.at[didx_v.at[0]],
                              ssem).wait()

    for k in range(LOOKAHEAD):
        pltpu.async_copy(tab_hbm.at[sidx_v.at[k]], rows_v.at[k], gsem)
    for i in range(LOOKAHEAD):
        wait_g(i)
        pltpu.async_copy(rows_v.at[i], acc_sp.at[didx_v.at[i]], ssem,
                         add=True)
        j = i + LOOKAHEAD
        pltpu.async_copy(tab_hbm.at[sidx_v.at[j]], rows_v.at[j % NBUF], gsem)

    def body(i, carry):
        b = i % NBUF
        wait_g(b)
        pltpu.async_copy(rows_v.at[b], acc_sp.at[didx_v.at[i]], ssem,
                         add=True)
        wait_s()
        j = i + LOOKAHEAD
        pltpu.async_copy(tab_hbm.at[sidx_v.at[j]], rows_v.at[j % NBUF], gsem)
        return carry

    lax.fori_loop(LOOKAHEAD, NCHUNK - LOOKAHEAD, body, 0)

    for i in range(NCHUNK - LOOKAHEAD, NCHUNK):
        wait_g(i % NBUF)
        pltpu.async_copy(rows_v.at[i % NBUF], acc_sp.at[didx_v.at[i]], ssem,
                         add=True)
    for _ in range(2 * LOOKAHEAD):
        wait_s()
    plsc.subcore_barrier()
    pltpu.sync_copy(acc_sp.at[pl.ds(s * RPW, RPW)],
                    out_hbm.at[c, pl.ds(s * RPW, RPW)])


_seg_kernel = pl.kernel(
    _seg_body,
    out_type=jax.ShapeDtypeStruct((NC, NPAD, H), jnp.bfloat16),
    mesh=_MESH,
    compiler_params=_SC_PARAMS,
    scratch_types=[
        pltpu.VMEM_SHARED((NPAD, H), jnp.bfloat16),
        pltpu.VMEM((NCHUNK, CH), jnp.int32),
        pltpu.VMEM((NCHUNK, CH), jnp.int32),
        pltpu.VMEM((NBUF, CH, H), jnp.bfloat16),
        pltpu.SemaphoreType.DMA,
        pltpu.SemaphoreType.DMA,
    ],
)


def _tc1_body(x_ref, w1_ref, degp_ref, ht_ref, dis_ref):
    deg2 = degp_ref[0][:, 0:2] + degp_ref[1][:, 0:2] + 1.0
    r = lax.broadcasted_iota(jnp.int32, (NPAIR, 2), 0)
    j = lax.broadcasted_iota(jnp.int32, (NPAIR, 2), 1)
    dis2 = jnp.where(r + j * NPAIR < N, lax.rsqrt(deg2), 0.0)
    dis_top = jnp.broadcast_to(dis2[:, 0:1], (NPAIR, H))
    dis_bot = jnp.broadcast_to(dis2[:, 1:2], (NPAIR, H))
    w1 = w1_ref[...]
    h_top = jnp.dot(x_ref[:NPAIR, :], w1, preferred_element_type=jnp.float32)
    h_bot = jnp.dot(x_ref[NPAIR:, :], w1, preferred_element_type=jnp.float32)
    ht = jnp.concatenate([h_top * dis_top, h_bot * dis_bot], axis=1)
    ht_ref[...] = ht.astype(jnp.bfloat16)
    dis_ref[...] = jnp.concatenate([dis_top, dis_bot], axis=1)


def _tc2_body(acc_ref, ht1_ref, dis_ref, b1_ref, w2_ref, ht2_ref):
    dis = dis_ref[...]
    agg = (acc_ref[0].astype(jnp.float32) + acc_ref[1].astype(jnp.float32)
           + ht1_ref[...].astype(jnp.float32)) * dis + b1_ref[...]
    a1 = jnp.maximum(agg, 0.0)
    w2 = w2_ref[...]
    m_ev = jnp.dot(a1[:, :H], w2, preferred_element_type=jnp.float32)
    m_od = jnp.dot(a1[:, H:], w2, preferred_element_type=jnp.float32)
    ht2 = jnp.concatenate([m_ev, m_od], axis=1) * dis
    ht2_ref[...] = ht2.astype(jnp.bfloat16)


def _tc3_body(acc_ref, ht2_ref, dis_ref, b2_ref, wc1_ref, bc1_ref,
              wc2_ref, bc2_ref, out_ref):
    contrib = (acc_ref[0].astype(jnp.float32) + acc_ref[1].astype(jnp.float32)
               + ht2_ref[...].astype(jnp.float32)) * dis_ref[...]
    srow = jnp.sum(contrib, axis=0, keepdims=True)
    g = (srow[:, :H] + srow[:, H:]) * (1.0 / N) + b2_ref[...]
    z = jnp.maximum(
        jnp.dot(g, wc1_ref[...], preferred_element_type=jnp.float32)
        + bc1_ref[...], 0.0)
    out_ref[...] = (jnp.dot(z, wc2_ref[...],
                            preferred_element_type=jnp.float32)
                    + bc2_ref[...])


def _tc_call(body, out_shape):
    return pl.pallas_call(body, out_shape=out_shape)


@jax.jit
def kernel(x, edge_index, W1, b1, W2, b2, Wc1, bc1, Wc2, bc2):
    pad = (N + (jnp.arange(EPAD - E, dtype=jnp.int32) % (NPAD - N)))
    dst = jnp.concatenate([edge_index[1].astype(jnp.int32), pad])
    dst_p = jnp.where(dst < NPAIR, 2 * dst,
                      2 * dst - (NPAD - 1)).reshape(EPAD // CH, CH)
    src = jnp.concatenate([edge_index[0].astype(jnp.int32), pad])
    src_p = jnp.where(src < NPAIR, 2 * src,
                      2 * src - (NPAD - 1)).reshape(EPAD // CH, CH)
    x_p = jnp.pad(x, ((0, NPAD - N), (0, 0)))

    zeros_n = jnp.zeros((NPAD,), jnp.float32)
    zeros_nh = jnp.zeros((NPAD, H), jnp.bfloat16)

    deg_parts = _deg_kernel(dst_p, zeros_n, jnp.asarray(_MIDX))

    ht1, dis = _tc_call(
        _tc1_body,
        [jax.ShapeDtypeStruct((NPAIR, 2 * H), jnp.bfloat16),
         jax.ShapeDtypeStruct((NPAIR, 2 * H), jnp.float32)],
    )(x_p, W1, deg_parts.reshape(NC, NPAIR, 128))

    acc1 = _seg_kernel(ht1.reshape(NPAD, H), src_p, dst_p, zeros_nh)

    ht2 = _tc_call(
        _tc2_body, jax.ShapeDtypeStruct((NPAIR, 2 * H), jnp.bfloat16)
    )(acc1.reshape(NC, NPAIR, 2 * H), ht1, dis,
      jnp.concatenate([b1, b1]).reshape(1, 2 * H), W2)

    acc2 = _seg_kernel(ht2.reshape(NPAD, H), src_p, dst_p, zeros_nh)

    logits = _tc_call(
        _tc3_body, jax.ShapeDtypeStruct((1, 2), jnp.float32)
    )(acc2.reshape(NC, NPAIR, 2 * H), ht2, dis, b2.reshape(1, H),
      Wc1, bc1.reshape(1, H // 2), Wc2, bc2.reshape(1, 2))
    return logits


# speedup vs baseline: 1.1170x; 1.1170x over previous
import functools


import jax
import jax.numpy as jnp
from jax import lax
from jax.experimental import pallas as pl
from jax.experimental.pallas import tpu as pltpu
from jax.experimental.pallas import tpu_sc as plsc

N = 10000
NPAD = 10240
E = 320000
EPAD = 327680
D = 128
H = 64

NC = 2
NS = 16
NW = NC * NS
EPW = EPAD // NW
CH = 128
NCHUNK = EPW // CH
RPW = NPAD // NS
NBUF = 8
LOOKAHEAD = 4
NPAIR = NPAD // 2

_MESH = plsc.VectorSubcoreMesh(
    core_axis_name="c", subcore_axis_name="s", num_cores=NC, num_subcores=NS
)


def _deg_body(dst_hbm, zeros_hbm, out_hbm, deg_sp, didx_v, ones_v, ssem):
    c = lax.axis_index("c")
    s = lax.axis_index("s")
    for j in range(8):
        ones_v[pl.ds(j * 16, 16)] = jnp.ones((16,), jnp.float32)
    pltpu.sync_copy(dst_hbm.at[pl.ds((c * NS + s) * NCHUNK, NCHUNK)], didx_v)
    pltpu.sync_copy(zeros_hbm.at[pl.ds(s * RPW, RPW)],
                    deg_sp.at[pl.ds(s * RPW, RPW)])
    plsc.subcore_barrier()

    def body(i, carry):
        pltpu.async_copy(ones_v, deg_sp.at[didx_v.at[i]], ssem, add=True)
        return carry

    lax.fori_loop(0, NCHUNK, body, 0)

    def drain(i, carry):
        pltpu.make_async_copy(ones_v, deg_sp.at[didx_v.at[0]], ssem).wait()
        return carry

    lax.fori_loop(0, NCHUNK, drain, 0)
    plsc.subcore_barrier()
    pltpu.sync_copy(deg_sp.at[pl.ds(s * RPW, RPW)],
                    out_hbm.at[c, pl.ds(s * RPW, RPW)])


_SC_PARAMS = pltpu.CompilerParams(use_tc_tiling_on_sc=False)

_deg_kernel = pl.kernel(
    _deg_body,
    out_type=jax.ShapeDtypeStruct((NC, NPAD), jnp.float32),
    mesh=_MESH,
    compiler_params=_SC_PARAMS,
    scratch_types=[
        pltpu.VMEM_SHARED((NPAD,), jnp.float32),
        pltpu.VMEM((NCHUNK, CH), jnp.int32),
        pltpu.VMEM((CH,), jnp.float32),
        pltpu.SemaphoreType.DMA,
    ],
)


def _seg_body(tab_hbm, src_hbm, dst_hbm, zeros_hbm, out_hbm,
              acc_sp, sidx_v, didx_v, rows_v, gsem, ssem):
    c = lax.axis_index("c")
    s = lax.axis_index("s")
    w = c * NS + s
    pltpu.sync_copy(src_hbm.at[pl.ds(w * NCHUNK, NCHUNK)], sidx_v)
    pltpu.sync_copy(dst_hbm.at[pl.ds(w * NCHUNK, NCHUNK)], didx_v)
    pltpu.sync_copy(zeros_hbm.at[pl.ds(s * RPW, RPW)],
                    acc_sp.at[pl.ds(s * RPW, RPW)])
    plsc.subcore_barrier()

    def wait_g(b):
        pltpu.make_async_copy(tab_hbm.at[sidx_v.at[0]], rows_v.at[b],
                              gsem).wait()

    def wait_s():
        pltpu.make_async_copy(rows_v.at[0], acc_sp.at[didx_v.at[0]],
                              ssem).wait()

    for k in range(LOOKAHEAD):
        pltpu.async_copy(tab_hbm.at[sidx_v.at[k]], rows_v.at[k], gsem)
    for i in range(LOOKAHEAD):
        wait_g(i)
        pltpu.async_copy(rows_v.at[i], acc_sp.at[didx_v.at[i]], ssem,
                         add=True)
        j = i + LOOKAHEAD
        pltpu.async_copy(tab_hbm.at[sidx_v.at[j]], rows_v.at[j % NBUF], gsem)

    def body(i, carry):
        b = i % NBUF
        wait_g(b)
        pltpu.async_copy(rows_v.at[b], acc_sp.at[didx_v.at[i]], ssem,
                         add=True)
        wait_s()
        j = i + LOOKAHEAD
        pltpu.async_copy(tab_hbm.at[sidx_v.at[j]], rows_v.at[j % NBUF], gsem)
        return carry

    lax.fori_loop(LOOKAHEAD, NCHUNK - LOOKAHEAD, body, 0)

    for i in range(NCHUNK - LOOKAHEAD, NCHUNK):
        wait_g(i % NBUF)
        pltpu.async_copy(rows_v.at[i % NBUF], acc_sp.at[didx_v.at[i]], ssem,
                         add=True)
    for _ in range(2 * LOOKAHEAD):
        wait_s()
    plsc.subcore_barrier()
    pltpu.sync_copy(acc_sp.at[pl.ds(s * RPW, RPW)],
                    out_hbm.at[c, pl.ds(s * RPW, RPW)])


_seg_kernel = pl.kernel(
    _seg_body,
    out_type=jax.ShapeDtypeStruct((NC, NPAD, H), jnp.bfloat16),
    mesh=_MESH,
    compiler_params=_SC_PARAMS,
    scratch_types=[
        pltpu.VMEM_SHARED((NPAD, H), jnp.bfloat16),
        pltpu.VMEM((NCHUNK, CH), jnp.int32),
        pltpu.VMEM((NCHUNK, CH), jnp.int32),
        pltpu.VMEM((NBUF, CH, H), jnp.bfloat16),
        pltpu.SemaphoreType.DMA,
        pltpu.SemaphoreType.DMA,
    ],
)


def _tc1_body(x_ref, w1_ref, degp_ref, ht_ref, dis_ref):
    deg2 = degp_ref[0] + degp_ref[1] + 1.0
    r = lax.broadcasted_iota(jnp.int32, (NPAIR, 2), 0)
    j = lax.broadcasted_iota(jnp.int32, (NPAIR, 2), 1)
    dis2 = jnp.where(r + j * NPAIR < N, lax.rsqrt(deg2), 0.0)
    dis_top = jnp.broadcast_to(dis2[:, 0:1], (NPAIR, H))
    dis_bot = jnp.broadcast_to(dis2[:, 1:2], (NPAIR, H))
    w1 = w1_ref[...]
    h_top = jnp.dot(x_ref[:NPAIR, :], w1, preferred_element_type=jnp.float32)
    h_bot = jnp.dot(x_ref[NPAIR:, :], w1, preferred_element_type=jnp.float32)
    ht = jnp.concatenate([h_top * dis_top, h_bot * dis_bot], axis=1)
    ht_ref[...] = ht.astype(jnp.bfloat16)
    dis_ref[...] = jnp.concatenate([dis_top, dis_bot], axis=1)


def _tc2_body(acc_ref, ht1_ref, dis_ref, b1_ref, w2_ref, ht2_ref):
    dis = dis_ref[...]
    agg = (acc_ref[0].astype(jnp.float32) + acc_ref[1].astype(jnp.float32)
           + ht1_ref[...].astype(jnp.float32)) * dis + b1_ref[...]
    a1 = jnp.maximum(agg, 0.0)
    w2 = w2_ref[...]
    m_ev = jnp.dot(a1[:, :H], w2, preferred_element_type=jnp.float32)
    m_od = jnp.dot(a1[:, H:], w2, preferred_element_type=jnp.float32)
    ht2 = jnp.concatenate([m_ev, m_od], axis=1) * dis
    ht2_ref[...] = ht2.astype(jnp.bfloat16)


def _tc3_body(acc_ref, ht2_ref, dis_ref, b2_ref, wc1_ref, bc1_ref,
              wc2_ref, bc2_ref, out_ref):
    contrib = (acc_ref[0].astype(jnp.float32) + acc_ref[1].astype(jnp.float32)
               + ht2_ref[...].astype(jnp.float32)) * dis_ref[...]
    srow = jnp.sum(contrib, axis=0, keepdims=True)
    g = (srow[:, :H] + srow[:, H:]) * (1.0 / N) + b2_ref[...]
    z = jnp.maximum(
        jnp.dot(g, wc1_ref[...], preferred_element_type=jnp.float32)
        + bc1_ref[...], 0.0)
    out_ref[...] = (jnp.dot(z, wc2_ref[...],
                            preferred_element_type=jnp.float32)
                    + bc2_ref[...])


def _tc_call(body, out_shape):
    return pl.pallas_call(body, out_shape=out_shape)


@jax.jit
def kernel(x, edge_index, W1, b1, W2, b2, Wc1, bc1, Wc2, bc2):
    pad = (N + (jnp.arange(EPAD - E, dtype=jnp.int32) % (NPAD - N)))
    dst = jnp.concatenate([edge_index[1].astype(jnp.int32), pad])
    dst_p = jnp.where(dst < NPAIR, 2 * dst,
                      2 * dst - (NPAD - 1)).reshape(EPAD // CH, CH)
    src = jnp.concatenate([edge_index[0].astype(jnp.int32), pad])
    src_p = jnp.where(src < NPAIR, 2 * src,
                      2 * src - (NPAD - 1)).reshape(EPAD // CH, CH)
    x_p = jnp.pad(x, ((0, NPAD - N), (0, 0)))

    zeros_n = jnp.zeros((NPAD,), jnp.float32)
    zeros_nh = jnp.zeros((NPAD, H), jnp.bfloat16)

    deg_parts = _deg_kernel(dst_p, zeros_n)

    ht1, dis = _tc_call(
        _tc1_body,
        [jax.ShapeDtypeStruct((NPAIR, 2 * H), jnp.bfloat16),
         jax.ShapeDtypeStruct((NPAIR, 2 * H), jnp.float32)],
    )(x_p, W1, deg_parts.reshape(NC, NPAIR, 2))

    acc1 = _seg_kernel(ht1.reshape(NPAD, H), src_p, dst_p, zeros_nh)

    ht2 = _tc_call(
        _tc2_body, jax.ShapeDtypeStruct((NPAIR, 2 * H), jnp.bfloat16)
    )(acc1.reshape(NC, NPAIR, 2 * H), ht1, dis,
      jnp.concatenate([b1, b1]).reshape(1, 2 * H), W2)

    acc2 = _seg_kernel(ht2.reshape(NPAD, H), src_p, dst_p, zeros_nh)

    logits = _tc_call(
        _tc3_body, jax.ShapeDtypeStruct((1, 2), jnp.float32)
    )(acc2.reshape(NC, NPAIR, 2 * H), ht2, dis, b2.reshape(1, H),
      Wc1, bc1.reshape(1, H // 2), Wc2, bc2.reshape(1, 2))
    return logits


# speedup vs baseline: 1.1211x; 1.0037x over previous
import functools


import jax
import jax.numpy as jnp
from jax import lax
from jax.experimental import pallas as pl
from jax.experimental.pallas import tpu as pltpu
from jax.experimental.pallas import tpu_sc as plsc

N = 10000
NPAD = 10240
E = 320000
EPAD = 327680
D = 128
H = 64

NC = 2
NS = 16
NW = NC * NS
EPW = EPAD // NW
CH = 128
NCHUNK = EPW // CH
RPW = NPAD // NS
NBUF = 12
LOOKAHEAD = 6
NPAIR = NPAD // 2

_MESH = plsc.VectorSubcoreMesh(
    core_axis_name="c", subcore_axis_name="s", num_cores=NC, num_subcores=NS
)


def _deg_body(dst_hbm, zeros_hbm, out_hbm, deg_sp, didx_v, ones_v, ssem):
    c = lax.axis_index("c")
    s = lax.axis_index("s")
    for j in range(8):
        ones_v[pl.ds(j * 16, 16)] = jnp.ones((16,), jnp.float32)
    pltpu.sync_copy(dst_hbm.at[pl.ds((c * NS + s) * NCHUNK, NCHUNK)], didx_v)
    pltpu.sync_copy(zeros_hbm.at[pl.ds(s * RPW, RPW)],
                    deg_sp.at[pl.ds(s * RPW, RPW)])
    plsc.subcore_barrier()

    def body(i, carry):
        pltpu.async_copy(ones_v, deg_sp.at[didx_v.at[i]], ssem, add=True)
        return carry

    lax.fori_loop(0, NCHUNK, body, 0)

    def drain(i, carry):
        pltpu.make_async_copy(ones_v, deg_sp.at[didx_v.at[0]], ssem).wait()
        return carry

    lax.fori_loop(0, NCHUNK, drain, 0)
    plsc.subcore_barrier()
    pltpu.sync_copy(deg_sp.at[pl.ds(s * RPW, RPW)],
                    out_hbm.at[c, pl.ds(s * RPW, RPW)])


_SC_PARAMS = pltpu.CompilerParams(use_tc_tiling_on_sc=False)

_deg_kernel = pl.kernel(
    _deg_body,
    out_type=jax.ShapeDtypeStruct((NC, NPAD), jnp.float32),
    mesh=_MESH,
    compiler_params=_SC_PARAMS,
    scratch_types=[
        pltpu.VMEM_SHARED((NPAD,), jnp.float32),
        pltpu.VMEM((NCHUNK, CH), jnp.int32),
        pltpu.VMEM((CH,), jnp.float32),
        pltpu.SemaphoreType.DMA,
    ],
)


def _seg_body(tab_hbm, src_hbm, dst_hbm, zeros_hbm, out_hbm,
              acc_sp, sidx_v, didx_v, rows_v, gsem, ssem):
    c = lax.axis_index("c")
    s = lax.axis_index("s")
    w = c * NS + s
    pltpu.sync_copy(src_hbm.at[pl.ds(w * NCHUNK, NCHUNK)], sidx_v)
    pltpu.sync_copy(dst_hbm.at[pl.ds(w * NCHUNK, NCHUNK)], didx_v)
    pltpu.sync_copy(zeros_hbm.at[pl.ds(s * RPW, RPW)],
                    acc_sp.at[pl.ds(s * RPW, RPW)])
    plsc.subcore_barrier()

    def wait_g(b):
        pltpu.make_async_copy(tab_hbm.at[sidx_v.at[0]], rows_v.at[b],
                              gsem).wait()

    def wait_s():
        pltpu.make_async_copy(rows_v.at[0], acc_sp.at[didx_v.at[0]],
                              ssem).wait()

    for k in range(LOOKAHEAD):
        pltpu.async_copy(tab_hbm.at[sidx_v.at[k]], rows_v.at[k], gsem)
    for i in range(LOOKAHEAD):
        wait_g(i)
        pltpu.async_copy(rows_v.at[i], acc_sp.at[didx_v.at[i]], ssem,
                         add=True)
        j = i + LOOKAHEAD
        pltpu.async_copy(tab_hbm.at[sidx_v.at[j]], rows_v.at[j % NBUF], gsem)

    def body(i, carry):
        b = i % NBUF
        wait_g(b)
        pltpu.async_copy(rows_v.at[b], acc_sp.at[didx_v.at[i]], ssem,
                         add=True)
        wait_s()
        j = i + LOOKAHEAD
        pltpu.async_copy(tab_hbm.at[sidx_v.at[j]], rows_v.at[j % NBUF], gsem)
        return carry

    lax.fori_loop(LOOKAHEAD, NCHUNK - LOOKAHEAD, body, 0)

    for i in range(NCHUNK - LOOKAHEAD, NCHUNK):
        wait_g(i % NBUF)
        pltpu.async_copy(rows_v.at[i % NBUF], acc_sp.at[didx_v.at[i]], ssem,
                         add=True)
    for _ in range(2 * LOOKAHEAD):
        wait_s()
    plsc.subcore_barrier()
    pltpu.sync_copy(acc_sp.at[pl.ds(s * RPW, RPW)],
                    out_hbm.at[c, pl.ds(s * RPW, RPW)])


_seg_kernel = pl.kernel(
    _seg_body,
    out_type=jax.ShapeDtypeStruct((NC, NPAD, H), jnp.bfloat16),
    mesh=_MESH,
    compiler_params=_SC_PARAMS,
    scratch_types=[
        pltpu.VMEM_SHARED((NPAD, H), jnp.bfloat16),
        pltpu.VMEM((NCHUNK, CH), jnp.int32),
        pltpu.VMEM((NCHUNK, CH), jnp.int32),
        pltpu.VMEM((NBUF, CH, H), jnp.bfloat16),
        pltpu.SemaphoreType.DMA,
        pltpu.SemaphoreType.DMA,
    ],
)


def _tc1_body(x_ref, w1_ref, degp_ref, ht_ref, dis_ref):
    deg2 = degp_ref[0] + degp_ref[1] + 1.0
    r = lax.broadcasted_iota(jnp.int32, (NPAIR, 2), 0)
    j = lax.broadcasted_iota(jnp.int32, (NPAIR, 2), 1)
    dis2 = jnp.where(r + j * NPAIR < N, lax.rsqrt(deg2), 0.0)
    dis_top = jnp.broadcast_to(dis2[:, 0:1], (NPAIR, H))
    dis_bot = jnp.broadcast_to(dis2[:, 1:2], (NPAIR, H))
    w1 = w1_ref[...]
    h_top = jnp.dot(x_ref[:NPAIR, :], w1, preferred_element_type=jnp.float32)
    h_bot = jnp.dot(x_ref[NPAIR:, :], w1, preferred_element_type=jnp.float32)
    ht = jnp.concatenate([h_top * dis_top, h_bot * dis_bot], axis=1)
    ht_ref[...] = ht.astype(jnp.bfloat16)
    dis_ref[...] = jnp.concatenate([dis_top, dis_bot], axis=1)


def _tc2_body(acc_ref, ht1_ref, dis_ref, b1_ref, w2_ref, ht2_ref):
    dis = dis_ref[...]
    agg = (acc_ref[0].astype(jnp.float32) + acc_ref[1].astype(jnp.float32)
           + ht1_ref[...].astype(jnp.float32)) * dis + b1_ref[...]
    a1 = jnp.maximum(agg, 0.0)
    w2 = w2_ref[...]
    m_ev = jnp.dot(a1[:, :H], w2, preferred_element_type=jnp.float32)
    m_od = jnp.dot(a1[:, H:], w2, preferred_element_type=jnp.float32)
    ht2 = jnp.concatenate([m_ev, m_od], axis=1) * dis
    ht2_ref[...] = ht2.astype(jnp.bfloat16)


def _tc3_body(acc_ref, ht2_ref, dis_ref, b2_ref, wc1_ref, bc1_ref,
              wc2_ref, bc2_ref, out_ref):
    contrib = (acc_ref[0].astype(jnp.float32) + acc_ref[1].astype(jnp.float32)
               + ht2_ref[...].astype(jnp.float32)) * dis_ref[...]
    srow = jnp.sum(contrib, axis=0, keepdims=True)
    g = (srow[:, :H] + srow[:, H:]) * (1.0 / N) + b2_ref[...]
    z = jnp.maximum(
        jnp.dot(g, wc1_ref[...], preferred_element_type=jnp.float32)
        + bc1_ref[...], 0.0)
    out_ref[...] = (jnp.dot(z, wc2_ref[...],
                            preferred_element_type=jnp.float32)
                    + bc2_ref[...])


def _tc_call(body, out_shape):
    return pl.pallas_call(body, out_shape=out_shape)


@jax.jit
def kernel(x, edge_index, W1, b1, W2, b2, Wc1, bc1, Wc2, bc2):
    pad = (N + (jnp.arange(EPAD - E, dtype=jnp.int32) % (NPAD - N)))
    dst = jnp.concatenate([edge_index[1].astype(jnp.int32), pad])
    dst_p = jnp.where(dst < NPAIR, 2 * dst,
                      2 * dst - (NPAD - 1)).reshape(EPAD // CH, CH)
    src = jnp.concatenate([edge_index[0].astype(jnp.int32), pad])
    src_p = jnp.where(src < NPAIR, 2 * src,
                      2 * src - (NPAD - 1)).reshape(EPAD // CH, CH)
    x_p = jnp.pad(x, ((0, NPAD - N), (0, 0)))

    zeros_n = jnp.zeros((NPAD,), jnp.float32)
    zeros_nh = jnp.zeros((NPAD, H), jnp.bfloat16)

    deg_parts = _deg_kernel(dst_p, zeros_n)

    ht1, dis = _tc_call(
        _tc1_body,
        [jax.ShapeDtypeStruct((NPAIR, 2 * H), jnp.bfloat16),
         jax.ShapeDtypeStruct((NPAIR, 2 * H), jnp.float32)],
    )(x_p, W1, deg_parts.reshape(NC, NPAIR, 2))

    acc1 = _seg_kernel(ht1.reshape(NPAD, H), src_p, dst_p, zeros_nh)

    ht2 = _tc_call(
        _tc2_body, jax.ShapeDtypeStruct((NPAIR, 2 * H), jnp.bfloat16)
    )(acc1.reshape(NC, NPAIR, 2 * H), ht1, dis,
      jnp.concatenate([b1, b1]).reshape(1, 2 * H), W2)

    acc2 = _seg_kernel(ht2.reshape(NPAD, H), src_p, dst_p, zeros_nh)

    logits = _tc_call(
        _tc3_body, jax.ShapeDtypeStruct((1, 2), jnp.float32)
    )(acc2.reshape(NC, NPAIR, 2 * H), ht2, dis, b2.reshape(1, H),
      Wc1, bc1.reshape(1, H // 2), Wc2, bc2.reshape(1, 2))
    return logits
